# baseline (speedup 1.0000x reference)
import functools

import jax
import jax.numpy as jnp
from jax.experimental import pallas as pl
from jax.experimental.pallas import tpu as pltpu
from jax.experimental.pallas import tpu_sc as plsc

_K = 8


def _routing_body(NB, BS, E, pt_ref, lt_ref, w2_ref, ones_ref, slt_ref,
                  probs_t_ref, idx_t_ref, rank_t_ref, counts_ref, base_ref,
                  carry_ref):
    b = pl.program_id(0)

    @pl.when(b == 0)
    def _init():
        carry_ref[...] = jnp.zeros_like(carry_ref)

    x = pt_ref[...].T
    eiota = jax.lax.broadcasted_iota(jnp.int32, (E, BS), 0)
    hot = jnp.zeros((E, BS), jnp.float32)
    sels = []
    for k in range(_K):
        m = jnp.max(x, axis=0, keepdims=True)
        eqw = jnp.where(x == m, w2_ref[...], 0.0)
        s = jnp.dot(ones_ref[...], eqw,
                    preferred_element_type=jnp.float32)
        first = 127 - jax.lax.shift_right_logical(
            jax.lax.bitcast_convert_type(s, jnp.int32), 23)
        sel = eiota == first
        probs_t_ref[k, :] = m[0, :]
        idx_t_ref[k, :] = first[0, :]
        x = jnp.where(sel, jnp.float32(-1.0), x)
        hot = hot + sel.astype(jnp.float32)
        sels.append(sel)

    incl = jnp.dot(hot, lt_ref[...], preferred_element_type=jnp.float32)
    rank = incl - hot + carry_ref[...]
    for k in range(_K):
        rk = jnp.dot(ones_ref[...], jnp.where(sels[k], rank, 0.0),
                     preferred_element_type=jnp.float32,
                     precision=jax.lax.Precision.HIGHEST)
        rank_t_ref[k, :] = rk[0, :].astype(jnp.int32)

    total = carry_ref[...] + incl[:, BS - 1:BS]
    carry_ref[...] = total

    @pl.when(b == NB - 1)
    def _fin():
        counts_ref[...] = total.astype(jnp.int32)
        base_ref[...] = jnp.dot(slt_ref[...], total,
                                preferred_element_type=jnp.float32,
                                precision=jax.lax.Precision.HIGHEST
                                ).astype(jnp.int32)


def _routing_call(pt, lt, w2, ones_row, slt):
    T, E = pt.shape
    BS = 1024
    NB = T // BS
    body = functools.partial(_routing_body, NB, BS, E)
    return pl.pallas_call(
        body,
        grid=(NB,),
        in_specs=[
            pl.BlockSpec((BS, E), lambda b: (b, 0)),
            pl.BlockSpec((BS, BS), lambda b: (0, 0)),
            pl.BlockSpec((E, 1), lambda b: (0, 0)),
            pl.BlockSpec((1, E), lambda b: (0, 0)),
            pl.BlockSpec((E, E), lambda b: (0, 0)),
        ],
        out_specs=[
            pl.BlockSpec((_K, BS), lambda b: (0, b)),
            pl.BlockSpec((_K, BS), lambda b: (0, b)),
            pl.BlockSpec((_K, BS), lambda b: (0, b)),
            pl.BlockSpec((E, 1), lambda b: (0, 0)),
            pl.BlockSpec((E, 1), lambda b: (0, 0)),
        ],
        out_shape=[
            jax.ShapeDtypeStruct((_K, T), jnp.float32),
            jax.ShapeDtypeStruct((_K, T), jnp.int32),
            jax.ShapeDtypeStruct((_K, T), jnp.int32),
            jax.ShapeDtypeStruct((E, 1), jnp.int32),
            jax.ShapeDtypeStruct((E, 1), jnp.int32),
        ],
        scratch_shapes=[pltpu.VMEM((E, 1), jnp.float32)],
    )(pt, lt, w2, ones_row, slt)


def _dest_body(E, eidx_ref, rank_ref, base_ref, dest_ref):
    eidx = eidx_ref[...]
    acc = jnp.zeros_like(eidx)
    for e in range(E):
        acc = jnp.where(eidx == e, base_ref[e, 0], acc)
    dest_ref[...] = rank_ref[...] + acc


def _dest_call(eidx_t, rank_t, base):
    T = eidx_t.shape[1]
    E = base.shape[0]
    BSB = 2048
    NBB = T // BSB
    return pl.pallas_call(
        functools.partial(_dest_body, E),
        grid=(NBB,),
        in_specs=[
            pl.BlockSpec((_K, BSB), lambda b: (0, b)),
            pl.BlockSpec((_K, BSB), lambda b: (0, b)),
            pl.BlockSpec((E, 1), lambda b: (0, 0)),
        ],
        out_specs=pl.BlockSpec((_K, BSB), lambda b: (0, b)),
        out_shape=jax.ShapeDtypeStruct((_K, T), jnp.int32),
    )(eidx_t, rank_t, base)


def _make_permute(T, H, NT):
    NW = 32
    TPW = T // NW
    NCH = TPW // NT
    NP = NCH // 2
    mesh = plsc.VectorSubcoreMesh(core_axis_name="c", subcore_axis_name="s",
                                  num_cores=2, num_subcores=16)
    scratch = (
        [pltpu.VMEM((NT, H), jnp.float32),
         pltpu.VMEM((NT, H), jnp.float32),
         pltpu.VMEM((_K * TPW,), jnp.int32),
         pltpu.VMEM((_K * TPW,), jnp.float32),
         pltpu.VMEM((TPW,), jnp.int32)]
        + [pltpu.VMEM((NT,), jnp.int32) for _ in range(2 * _K)]
        + [pltpu.SemaphoreType.DMA] * 5
    )

    @functools.partial(
        pl.kernel,
        out_type=(
            jax.ShapeDtypeStruct((T * _K, H), jnp.float32),
            jax.ShapeDtypeStruct((T * _K,), jnp.float32),
            jax.ShapeDtypeStruct((T * _K,), jnp.int32),
        ),
        mesh=mesh,
        scratch_types=scratch,
    )
    def permute(hidden, dest_f, probs_f, tokids,
                out_rows, out_probs, out_ids,
                rows_a, rows_b, dw, pw, tw, *rest):
        didx_a = rest[:_K]
        didx_b = rest[_K:2 * _K]
        sem_pre, sem_la, sem_lb, sem_sa, sem_sb = rest[2 * _K:]
        wid = jax.lax.axis_index("c") * 16 + jax.lax.axis_index("s")
        w0 = wid * TPW

        pre = [pltpu.async_copy(tokids.at[pl.ds(w0, TPW)], tw, sem_pre)]
        for k in range(_K):
            pre.append(pltpu.async_copy(
                dest_f.at[pl.ds(k * T + w0, TPW)],
                dw.at[pl.ds(k * TPW, TPW)], sem_pre))
            pre.append(pltpu.async_copy(
                probs_f.at[pl.ds(k * T + w0, TPW)],
                pw.at[pl.ds(k * TPW, TPW)], sem_pre))
        for cp in pre:
            cp.wait()

        def load_rows(c, buf, sem):
            return pltpu.async_copy(hidden.at[pl.ds(w0 + c * NT, NT)], buf, sem)

        def fill_didx(c, didx):
            for k in range(_K):
                for j in range(NT // 16):
                    s = pl.ds(j * 16, 16)
                    didx[k][s] = dw[pl.ds(k * TPW + c * NT + j * 16, 16)]

        def fire_scatters(c, buf, didx, sem):
            cps = []
            for k in range(_K):
                cps.append(pltpu.async_copy(buf, out_rows.at[didx[k]], sem))
                cps.append(pltpu.async_copy(
                    pw.at[pl.ds(k * TPW + c * NT, NT)],
                    out_probs.at[didx[k]], sem))
                cps.append(pltpu.async_copy(
                    tw.at[pl.ds(c * NT, NT)], out_ids.at[didx[k]], sem))
            return cps

        def wait_scatters(buf, didx, sem):
            for k in range(_K):
                pltpu.make_async_copy(buf, out_rows.at[didx[k]], sem).wait()
                pltpu.make_async_copy(
                    pw.at[pl.ds(0, NT)], out_probs.at[didx[k]], sem).wait()
                pltpu.make_async_copy(
                    tw.at[pl.ds(0, NT)], out_ids.at[didx[k]], sem).wait()

        load_rows(0, rows_a, sem_la)

        def pair(g, carry):
            c0 = 2 * g
            c1 = c0 + 1
            fill_didx(c0, didx_a)
            pltpu.make_async_copy(
                hidden.at[pl.ds(0, NT)], rows_a, sem_la).wait()

            @pl.when(g > 0)
            def _():
                wait_scatters(rows_b, didx_b, sem_sb)

            sts0 = fire_scatters(c0, rows_a, didx_a, sem_sa)
            load_rows(c1, rows_b, sem_lb)
            fill_didx(c1, didx_b)
            pltpu.make_async_copy(
                hidden.at[pl.ds(0, NT)], rows_b, sem_lb).wait()
            for cp in sts0:
                cp.wait()
            fire_scatters(c1, rows_b, didx_b, sem_sb)

            @pl.when(g < NP - 1)
            def _():
                load_rows(c0 + 2, rows_a, sem_la)

            return carry

        jax.lax.fori_loop(0, NP, pair, 0)
        wait_scatters(rows_b, didx_b, sem_sb)

    return permute


def kernel(hidden_states, token_probs):
    T, H = hidden_states.shape
    E = token_probs.shape[1]
    BS = 1024
    r = jnp.arange(BS, dtype=jnp.int32)
    lt = (r[:, None] <= r[None, :]).astype(jnp.float32)
    e = jnp.arange(E, dtype=jnp.int32)
    w2 = jax.lax.bitcast_convert_type(
        (127 - e) << 23, jnp.float32)[:, None]
    ones_row = jnp.ones((1, E), jnp.float32)
    slt = (e[:, None] > e[None, :]).astype(jnp.float32)

    probs_t, eidx_t, rank_t, counts, base = _routing_call(
        token_probs, lt, w2, ones_row, slt)
    dest_t = _dest_call(eidx_t, rank_t, base)
    tokids = jnp.arange(T, dtype=jnp.int32)
    permute = _make_permute(T, H, NT=32)
    out_rows, out_probs, out_ids = permute(
        hidden_states, dest_t.reshape(-1), probs_t.reshape(-1), tokids)
    return out_rows, out_probs, out_ids, counts.reshape(-1)


# speedup vs baseline: 1.0079x; 1.0079x over previous
import functools

import jax
import jax.numpy as jnp
from jax.experimental import pallas as pl
from jax.experimental.pallas import tpu as pltpu
from jax.experimental.pallas import tpu_sc as plsc

_K = 8


def _routing_body(NB, BS, NBB, BSB, E, pt_ref, lt_ref, w2_ref, ones_ref,
                  slt_ref, probs_t_ref, dest_ref, counts_ref,
                  carry_ref, idx_scr, rank_scr, base_scr):
    b = pl.program_id(0)

    @pl.when(b == 0)
    def _init():
        carry_ref[...] = jnp.zeros_like(carry_ref)

    @pl.when(b < NB)
    def _routing_pass():
        x = pt_ref[...]
        eiota = jax.lax.broadcasted_iota(jnp.int32, (E, BS), 0)
        hot = jnp.zeros((E, BS), jnp.float32)
        sels = []
        for k in range(_K):
            m = jnp.max(x, axis=0, keepdims=True)
            eqw = jnp.where(x == m, w2_ref[...], 0.0)
            s = jnp.dot(ones_ref[...], eqw,
                        preferred_element_type=jnp.float32)
            first = 127 - jax.lax.shift_right_logical(
                jax.lax.bitcast_convert_type(s, jnp.int32), 23)
            sel = eiota == first
            probs_t_ref[k, :] = m[0, :]
            idx_scr[k, pl.ds(b * BS, BS)] = first[0, :]
            x = jnp.where(sel, jnp.float32(-1.0), x)
            hot = hot + sel.astype(jnp.float32)
            sels.append(sel)

        incl = jnp.dot(hot, lt_ref[...], preferred_element_type=jnp.float32)
        rank = incl - hot + carry_ref[...]
        for k in range(_K):
            rk = jnp.dot(ones_ref[...], jnp.where(sels[k], rank, 0.0),
                         preferred_element_type=jnp.float32,
                         precision=jax.lax.Precision.HIGHEST)
            rank_scr[k, pl.ds(b * BS, BS)] = rk[0, :].astype(jnp.int32)

        total = carry_ref[...] + incl[:, BS - 1:BS]
        carry_ref[...] = total

        @pl.when(b == NB - 1)
        def _fin():
            counts_ref[...] = total.astype(jnp.int32)
            base_scr[...] = jnp.dot(slt_ref[...], total,
                                    preferred_element_type=jnp.float32,
                                    precision=jax.lax.Precision.HIGHEST
                                    ).astype(jnp.int32)

    @pl.when(b >= NB)
    def _dest_pass():
        off = (b - NB) * BSB
        ei = idx_scr[:, pl.ds(off, BSB)]
        rk = rank_scr[:, pl.ds(off, BSB)]
        acc = jnp.zeros_like(ei)
        for e in range(E):
            acc = jnp.where(ei == e, base_scr[e, 0], acc)
        dest_ref[...] = rk + acc


def _routing_call(pt, lt, w2, ones_row, slt):
    E, T = pt.shape
    BS = 1024
    NB = T // BS
    BSB = 2048
    NBB = T // BSB
    body = functools.partial(_routing_body, NB, BS, NBB, BSB, E)
    return pl.pallas_call(
        body,
        grid=(NB + NBB,),
        in_specs=[
            pl.BlockSpec((E, BS), lambda b: (0, jnp.minimum(b, NB - 1))),
            pl.BlockSpec((BS, BS), lambda b: (0, 0)),
            pl.BlockSpec((E, 1), lambda b: (0, 0)),
            pl.BlockSpec((1, E), lambda b: (0, 0)),
            pl.BlockSpec((E, E), lambda b: (0, 0)),
        ],
        out_specs=[
            pl.BlockSpec((_K, BS), lambda b: (0, jnp.minimum(b, NB - 1))),
            pl.BlockSpec((_K, BSB), lambda b: (0, jnp.maximum(b - NB, 0))),
            pl.BlockSpec((E, 1), lambda b: (0, 0)),
        ],
        out_shape=[
            jax.ShapeDtypeStruct((_K, T), jnp.float32),
            jax.ShapeDtypeStruct((_K, T), jnp.int32),
            jax.ShapeDtypeStruct((E, 1), jnp.int32),
        ],
        scratch_shapes=[
            pltpu.VMEM((E, 1), jnp.float32),
            pltpu.VMEM((_K, T), jnp.int32),
            pltpu.VMEM((_K, T), jnp.int32),
            pltpu.VMEM((E, 1), jnp.int32),
        ],
    )(pt, lt, w2, ones_row, slt)


def _make_permute(T, H, NT):
    NW = 32
    TPW = T // NW
    NCH = TPW // NT
    NP = NCH // 2
    mesh = plsc.VectorSubcoreMesh(core_axis_name="c", subcore_axis_name="s",
                                  num_cores=2, num_subcores=16)
    scratch = (
        [pltpu.VMEM((NT, H), jnp.float32),
         pltpu.VMEM((NT, H), jnp.float32),
         pltpu.VMEM((_K * TPW,), jnp.int32),
         pltpu.VMEM((_K * TPW,), jnp.float32),
         pltpu.VMEM((TPW,), jnp.int32)]
        + [pltpu.VMEM((NT,), jnp.int32) for _ in range(2 * _K)]
        + [pltpu.SemaphoreType.DMA] * 5
    )

    @functools.partial(
        pl.kernel,
        out_type=(
            jax.ShapeDtypeStruct((T * _K, H), jnp.float32),
            jax.ShapeDtypeStruct((T * _K,), jnp.float32),
            jax.ShapeDtypeStruct((T * _K,), jnp.int32),
        ),
        mesh=mesh,
        scratch_types=scratch,
    )
    def permute(hidden, dest_f, probs_f, tokids,
                out_rows, out_probs, out_ids,
                rows_a, rows_b, dw, pw, tw, *rest):
        didx_a = rest[:_K]
        didx_b = rest[_K:2 * _K]
        sem_pre, sem_la, sem_lb, sem_sa, sem_sb = rest[2 * _K:]
        wid = jax.lax.axis_index("c") * 16 + jax.lax.axis_index("s")
        w0 = wid * TPW

        pre = [pltpu.async_copy(tokids.at[pl.ds(w0, TPW)], tw, sem_pre)]
        for k in range(_K):
            pre.append(pltpu.async_copy(
                dest_f.at[pl.ds(k * T + w0, TPW)],
                dw.at[pl.ds(k * TPW, TPW)], sem_pre))
            pre.append(pltpu.async_copy(
                probs_f.at[pl.ds(k * T + w0, TPW)],
                pw.at[pl.ds(k * TPW, TPW)], sem_pre))
        for cp in pre:
            cp.wait()

        def load_rows(c, buf, sem):
            return pltpu.async_copy(hidden.at[pl.ds(w0 + c * NT, NT)], buf, sem)

        def fill_didx(c, didx):
            for k in range(_K):
                for j in range(NT // 16):
                    s = pl.ds(j * 16, 16)
                    didx[k][s] = dw[pl.ds(k * TPW + c * NT + j * 16, 16)]

        def fire_scatters(c, buf, didx, sem):
            cps = []
            for k in range(_K):
                cps.append(pltpu.async_copy(buf, out_rows.at[didx[k]], sem))
                cps.append(pltpu.async_copy(
                    pw.at[pl.ds(k * TPW + c * NT, NT)],
                    out_probs.at[didx[k]], sem))
                cps.append(pltpu.async_copy(
                    tw.at[pl.ds(c * NT, NT)], out_ids.at[didx[k]], sem))
            return cps

        def wait_scatters(buf, didx, sem):
            for k in range(_K):
                pltpu.make_async_copy(buf, out_rows.at[didx[k]], sem).wait()
                pltpu.make_async_copy(
                    pw.at[pl.ds(0, NT)], out_probs.at[didx[k]], sem).wait()
                pltpu.make_async_copy(
                    tw.at[pl.ds(0, NT)], out_ids.at[didx[k]], sem).wait()

        load_rows(0, rows_a, sem_la)

        def pair(g, carry):
            c0 = 2 * g
            c1 = c0 + 1
            fill_didx(c0, didx_a)
            pltpu.make_async_copy(
                hidden.at[pl.ds(0, NT)], rows_a, sem_la).wait()

            @pl.when(g > 0)
            def _():
                wait_scatters(rows_b, didx_b, sem_sb)

            sts0 = fire_scatters(c0, rows_a, didx_a, sem_sa)
            load_rows(c1, rows_b, sem_lb)
            fill_didx(c1, didx_b)
            pltpu.make_async_copy(
                hidden.at[pl.ds(0, NT)], rows_b, sem_lb).wait()
            for cp in sts0:
                cp.wait()
            fire_scatters(c1, rows_b, didx_b, sem_sb)

            @pl.when(g < NP - 1)
            def _():
                load_rows(c0 + 2, rows_a, sem_la)

            return carry

        jax.lax.fori_loop(0, NP, pair, 0)
        wait_scatters(rows_b, didx_b, sem_sb)

    return permute


def kernel(hidden_states, token_probs):
    T, H = hidden_states.shape
    E = token_probs.shape[1]
    pt = token_probs.T

    BS = 1024
    r = jnp.arange(BS, dtype=jnp.int32)
    lt = (r[:, None] <= r[None, :]).astype(jnp.float32)
    e = jnp.arange(E, dtype=jnp.int32)
    w2 = jax.lax.bitcast_convert_type(
        (127 - e) << 23, jnp.float32)[:, None]
    ones_row = jnp.ones((1, E), jnp.float32)
    slt = (e[:, None] > e[None, :]).astype(jnp.float32)

    probs_t, dest_t, counts = _routing_call(pt, lt, w2, ones_row, slt)
    tokids = jnp.arange(T, dtype=jnp.int32)
    permute = _make_permute(T, H, NT=32)
    out_rows, out_probs, out_ids = permute(
        hidden_states, dest_t.reshape(-1), probs_t.reshape(-1), tokids)
    return out_rows, out_probs, out_ids, counts.reshape(-1)


# speedup vs baseline: 1.0080x; 1.0001x over previous
import functools

import jax
import jax.numpy as jnp
from jax.experimental import pallas as pl
from jax.experimental.pallas import tpu as pltpu
from jax.experimental.pallas import tpu_sc as plsc

_K = 8


def _routing_body(NB, BS, NBB, BSB, E, pt_ref, lt_ref, w2_ref, ones_ref,
                  slt_ref, probs_t_ref, dest_ref, counts_ref,
                  carry_ref, idx_scr, rank_scr, base_scr):
    b = pl.program_id(0)

    @pl.when(b == 0)
    def _init():
        carry_ref[...] = jnp.zeros_like(carry_ref)

    @pl.when(b < NB)
    def _routing_pass():
        x = pt_ref[...]
        eiota = jax.lax.broadcasted_iota(jnp.int32, (E, BS), 0)
        hot = jnp.zeros((E, BS), jnp.float32)
        sels = []
        for k in range(_K):
            m = jnp.max(x, axis=0, keepdims=True)
            eqw = jnp.where(x == m, w2_ref[...], 0.0)
            s = jnp.dot(ones_ref[...], eqw,
                        preferred_element_type=jnp.float32)
            first = 127 - jax.lax.shift_right_logical(
                jax.lax.bitcast_convert_type(s, jnp.int32), 23)
            sel = eiota == first
            probs_t_ref[k, :] = m[0, :]
            idx_scr[k, pl.ds(b * BS, BS)] = first[0, :]
            x = jnp.where(sel, jnp.float32(-1.0), x)
            hot = hot + sel.astype(jnp.float32)
            sels.append(sel)

        incl = jnp.dot(hot, lt_ref[...], preferred_element_type=jnp.float32)
        rank = incl - hot + carry_ref[...]
        for k in range(_K):
            rk = jnp.dot(ones_ref[...], jnp.where(sels[k], rank, 0.0),
                         preferred_element_type=jnp.float32,
                         precision=jax.lax.Precision.HIGHEST)
            rank_scr[k, pl.ds(b * BS, BS)] = rk[0, :].astype(jnp.int32)

        total = carry_ref[...] + incl[:, BS - 1:BS]
        carry_ref[...] = total

        @pl.when(b == NB - 1)
        def _fin():
            counts_ref[...] = total.astype(jnp.int32)
            base_scr[...] = jnp.dot(slt_ref[...], total,
                                    preferred_element_type=jnp.float32,
                                    precision=jax.lax.Precision.HIGHEST
                                    ).astype(jnp.int32)

    @pl.when(b >= NB)
    def _dest_pass():
        off = (b - NB) * BSB
        ei = idx_scr[:, pl.ds(off, BSB)]
        rk = rank_scr[:, pl.ds(off, BSB)]
        acc = jnp.zeros_like(ei)
        for e in range(E):
            acc = jnp.where(ei == e, base_scr[e, 0], acc)
        dest_ref[...] = rk + acc


def _routing_call(pt, lt, w2, ones_row, slt):
    E, T = pt.shape
    BS = 1024
    NB = T // BS
    BSB = 2048
    NBB = T // BSB
    body = functools.partial(_routing_body, NB, BS, NBB, BSB, E)
    return pl.pallas_call(
        body,
        grid=(NB + NBB,),
        in_specs=[
            pl.BlockSpec((E, BS), lambda b: (0, jnp.minimum(b, NB - 1))),
            pl.BlockSpec((BS, BS), lambda b: (0, 0)),
            pl.BlockSpec((E, 1), lambda b: (0, 0)),
            pl.BlockSpec((1, E), lambda b: (0, 0)),
            pl.BlockSpec((E, E), lambda b: (0, 0)),
        ],
        out_specs=[
            pl.BlockSpec((_K, BS), lambda b: (0, jnp.minimum(b, NB - 1))),
            pl.BlockSpec((_K, BSB), lambda b: (0, jnp.maximum(b - NB, 0))),
            pl.BlockSpec((E, 1), lambda b: (0, 0)),
        ],
        out_shape=[
            jax.ShapeDtypeStruct((_K, T), jnp.float32),
            jax.ShapeDtypeStruct((_K, T), jnp.int32),
            jax.ShapeDtypeStruct((E, 1), jnp.int32),
        ],
        scratch_shapes=[
            pltpu.VMEM((E, 1), jnp.float32),
            pltpu.VMEM((_K, T), jnp.int32),
            pltpu.VMEM((_K, T), jnp.int32),
            pltpu.VMEM((E, 1), jnp.int32),
        ],
    )(pt, lt, w2, ones_row, slt)


def _make_permute(T, H, NT):
    NW = 32
    TPW = T // NW
    NCH = TPW // NT
    NP = NCH // 2
    mesh = plsc.VectorSubcoreMesh(core_axis_name="c", subcore_axis_name="s",
                                  num_cores=2, num_subcores=16)
    scratch = (
        [pltpu.VMEM((NT, H), jnp.float32),
         pltpu.VMEM((NT, H), jnp.float32),
         pltpu.VMEM((_K * TPW,), jnp.int32),
         pltpu.VMEM((_K * TPW,), jnp.float32),
         pltpu.VMEM((TPW,), jnp.int32)]
        + [pltpu.VMEM((NT,), jnp.int32) for _ in range(2 * _K)]
        + [pltpu.SemaphoreType.DMA] * 5
    )

    @functools.partial(
        pl.kernel,
        out_type=(
            jax.ShapeDtypeStruct((T * _K, H), jnp.float32),
            jax.ShapeDtypeStruct((T * _K,), jnp.float32),
            jax.ShapeDtypeStruct((T * _K,), jnp.int32),
        ),
        mesh=mesh,
        scratch_types=scratch,
    )
    def permute(hidden, dest_f, probs_f, tokids,
                out_rows, out_probs, out_ids,
                rows_a, rows_b, dw, pw, tw, *rest):
        didx_a = rest[:_K]
        didx_b = rest[_K:2 * _K]
        sem_pre, sem_la, sem_lb, sem_sa, sem_sb = rest[2 * _K:]
        wid = jax.lax.axis_index("c") * 16 + jax.lax.axis_index("s")
        w0 = wid * TPW

        pre = [pltpu.async_copy(tokids.at[pl.ds(w0, TPW)], tw, sem_pre)]
        for k in range(_K):
            pre.append(pltpu.async_copy(
                dest_f.at[pl.ds(k * T + w0, TPW)],
                dw.at[pl.ds(k * TPW, TPW)], sem_pre))
            pre.append(pltpu.async_copy(
                probs_f.at[pl.ds(k * T + w0, TPW)],
                pw.at[pl.ds(k * TPW, TPW)], sem_pre))
        def load_rows(c, buf, sem):
            return pltpu.async_copy(hidden.at[pl.ds(w0 + c * NT, NT)], buf, sem)

        load_rows(0, rows_a, sem_la)
        for cp in pre:
            cp.wait()

        def fill_didx(c, didx):
            for k in range(_K):
                for j in range(NT // 16):
                    s = pl.ds(j * 16, 16)
                    didx[k][s] = dw[pl.ds(k * TPW + c * NT + j * 16, 16)]

        def fire_scatters(c, buf, didx, sem):
            cps = []
            for k in range(_K):
                cps.append(pltpu.async_copy(buf, out_rows.at[didx[k]], sem))
                cps.append(pltpu.async_copy(
                    pw.at[pl.ds(k * TPW + c * NT, NT)],
                    out_probs.at[didx[k]], sem))
                cps.append(pltpu.async_copy(
                    tw.at[pl.ds(c * NT, NT)], out_ids.at[didx[k]], sem))
            return cps

        def wait_scatters(buf, didx, sem):
            for k in range(_K):
                pltpu.make_async_copy(buf, out_rows.at[didx[k]], sem).wait()
                pltpu.make_async_copy(
                    pw.at[pl.ds(0, NT)], out_probs.at[didx[k]], sem).wait()
                pltpu.make_async_copy(
                    tw.at[pl.ds(0, NT)], out_ids.at[didx[k]], sem).wait()

        def pair(g, carry):
            c0 = 2 * g
            c1 = c0 + 1
            fill_didx(c0, didx_a)
            pltpu.make_async_copy(
                hidden.at[pl.ds(0, NT)], rows_a, sem_la).wait()

            @pl.when(g > 0)
            def _():
                wait_scatters(rows_b, didx_b, sem_sb)

            sts0 = fire_scatters(c0, rows_a, didx_a, sem_sa)
            load_rows(c1, rows_b, sem_lb)
            fill_didx(c1, didx_b)
            pltpu.make_async_copy(
                hidden.at[pl.ds(0, NT)], rows_b, sem_lb).wait()
            for cp in sts0:
                cp.wait()
            fire_scatters(c1, rows_b, didx_b, sem_sb)

            @pl.when(g < NP - 1)
            def _():
                load_rows(c0 + 2, rows_a, sem_la)

            return carry

        jax.lax.fori_loop(0, NP, pair, 0)
        wait_scatters(rows_b, didx_b, sem_sb)

    return permute


def kernel(hidden_states, token_probs):
    T, H = hidden_states.shape
    E = token_probs.shape[1]
    pt = token_probs.T

    BS = 1024
    r = jnp.arange(BS, dtype=jnp.int32)
    lt = (r[:, None] <= r[None, :]).astype(jnp.float32)
    e = jnp.arange(E, dtype=jnp.int32)
    w2 = jax.lax.bitcast_convert_type(
        (127 - e) << 23, jnp.float32)[:, None]
    ones_row = jnp.ones((1, E), jnp.float32)
    slt = (e[:, None] > e[None, :]).astype(jnp.float32)

    probs_t, dest_t, counts = _routing_call(pt, lt, w2, ones_row, slt)
    tokids = jnp.arange(T, dtype=jnp.int32)
    permute = _make_permute(T, H, NT=32)
    out_rows, out_probs, out_ids = permute(
        hidden_states, dest_t.reshape(-1), probs_t.reshape(-1), tokids)
    return out_rows, out_probs, out_ids, counts.reshape(-1)
